# Initial kernel scaffold; baseline (speedup 1.0000x reference)
import jax
import jax.numpy as jnp
from jax.experimental import pallas as pl


def kernel(x, adj_matrix, W_gat, att_src, att_dst, b_gat, W_ih, W_hh, b_ih, b_hh, W_conv, b_conv, W_out, b_out):
    raise NotImplementedError("write your pallas kernel here")


# speedup vs baseline: 2632.5876x; 2632.5876x over previous
import jax
import jax.numpy as jnp
from jax.experimental import pallas as pl


def _gat_kernel(x_ref, wgT_ref, asrc_ref, adst_ref, bg_ref, out_ref):
    xt = x_ref[0]
    h = jnp.dot(xt, wgT_ref[...], preferred_element_type=jnp.float32)
    a_d = jax.lax.dot_general(h, adst_ref[...], (((1,), (1,)), ((), ())),
                              preferred_element_type=jnp.float32)
    a_s = jax.lax.dot_general(asrc_ref[...], h, (((1,), (1,)), ((), ())),
                              preferred_element_type=jnp.float32)
    e = a_d + a_s
    e = jnp.where(e > 0, e, 0.2 * e)
    m = jnp.max(e, axis=1, keepdims=True)
    ex = jnp.exp(e - m)
    s = jnp.sum(ex, axis=1, keepdims=True)
    alpha = ex / (s + 1e-16)
    g = jnp.dot(alpha, h, preferred_element_type=jnp.float32) + bg_ref[...]
    out_ref[0] = g


def _tail_kernel(xs_ref, wihT_ref, whhT_ref, bih_ref, bhh_ref,
                 m2_ref, b2_ref, out_ref):
    T, N, H = xs_ref.shape
    h = jnp.zeros((N, 4), jnp.float32)
    hs = []
    for t in range(T):
        xt = xs_ref[t]
        gi = jnp.dot(xt, wihT_ref[...], preferred_element_type=jnp.float32) + bih_ref[...]
        gh = jnp.dot(h, whhT_ref[...], preferred_element_type=jnp.float32) + bhh_ref[...]
        r = jax.nn.sigmoid(gi[:, 0:4] + gh[:, 0:4])
        z = jax.nn.sigmoid(gi[:, 4:8] + gh[:, 4:8])
        n = jnp.tanh(gi[:, 8:12] + r * gh[:, 8:12])
        h = (1.0 - z) * n + z * h
        hs.append(h)
    zrow = jnp.zeros((1, 4), jnp.float32)
    blocks = []
    for c in range(T):
        hc = hs[c]
        blocks.append(jnp.concatenate([zrow, hc[:-1]], axis=0))
        blocks.append(hc)
        blocks.append(jnp.concatenate([hc[1:], zrow], axis=0))
    A = jnp.concatenate(blocks, axis=1)
    out_ref[...] = jnp.dot(A, m2_ref[...], preferred_element_type=jnp.float32) + b2_ref[...]


def kernel(x, adj_matrix, W_gat, att_src, att_dst, b_gat,
           W_ih, W_hh, b_ih, b_hh, W_conv, b_conv, W_out, b_out):
    B, T, N, Fin = x.shape
    H = W_gat.shape[0]
    O = W_conv.shape[0]
    x0 = x[0]

    Wt = jnp.transpose(W_conv, (1, 2, 3, 0))
    M = jnp.zeros((T, 3, 4, O, 4), jnp.float32)
    for w in range(4):
        for dx in range(3):
            wi = w - 1 + dx
            if 0 <= wi < 4:
                M = M.at[:, :, wi, :, w].set(Wt[:, :, dx, :])
    M2 = jnp.einsum('cdiow,kw->cdiok', M, W_out).reshape(T * 3 * 4, O * 2)
    bias2 = (b_conv[:, None] * jnp.sum(W_out, axis=1)[None, :]
             + b_out[None, :]).reshape(1, O * 2)

    G = pl.pallas_call(
        _gat_kernel,
        grid=(T,),
        in_specs=[
            pl.BlockSpec((1, N, Fin), lambda t: (t, 0, 0)),
            pl.BlockSpec((Fin, H), lambda t: (0, 0)),
            pl.BlockSpec((1, H), lambda t: (0, 0)),
            pl.BlockSpec((1, H), lambda t: (0, 0)),
            pl.BlockSpec((1, H), lambda t: (0, 0)),
        ],
        out_specs=pl.BlockSpec((1, N, H), lambda t: (t, 0, 0)),
        out_shape=jax.ShapeDtypeStruct((T, N, H), jnp.float32),
    )(x0, W_gat.T, att_src[None, :], att_dst[None, :], b_gat[None, :])

    Xs = G.reshape(N, T, H).transpose(1, 0, 2)

    F = pl.pallas_call(
        _tail_kernel,
        out_shape=jax.ShapeDtypeStruct((N, O * 2), jnp.float32),
    )(Xs, W_ih.T, W_hh.T, b_ih[None, :], b_hh[None, :], M2, bias2)

    return F.reshape(N, O, 2).transpose(1, 0, 2)[None]
